# Initial kernel scaffold; baseline (speedup 1.0000x reference)
import jax
import jax.numpy as jnp
from jax.experimental import pallas as pl


def kernel(hidden_states, gate_w, w1, w2, w3):
    raise NotImplementedError("write your pallas kernel here")


# speedup vs baseline: 1.5842x; 1.5842x over previous
import functools

import jax
import jax.numpy as jnp
from jax.experimental import pallas as pl
from jax.experimental.pallas import tpu as pltpu

H = 1024
FF = 2816
E = 8
FF_TILE = 1408
N_FT = FF // FF_TILE


def _moe_kernel(x_ref, gate_ref, w1_ref, w2_ref, w3_ref,
                out_ref, logits_ref, coef_ref):
    e = pl.program_id(0)
    f = pl.program_id(1)

    @pl.when((e == 0) & (f == 0))
    def _router():
        x = x_ref[...]
        logits = jnp.dot(x, gate_ref[...], preferred_element_type=jnp.float32)
        logits_ref[...] = logits
        idx = jax.lax.broadcasted_iota(jnp.int32, logits.shape, 1)
        v1 = jnp.max(logits, axis=1, keepdims=True)
        i1 = jnp.min(jnp.where(logits == v1, idx, E), axis=1, keepdims=True)
        oh1 = idx == i1
        masked = jnp.where(oh1, -jnp.inf, logits)
        v2 = jnp.max(masked, axis=1, keepdims=True)
        i2 = jnp.min(jnp.where(masked == v2, idx, E), axis=1, keepdims=True)
        oh2 = idx == i2
        p1 = 1.0 / (1.0 + jnp.exp(v2 - v1))
        p2 = 1.0 - p1
        coef_ref[...] = jnp.where(oh1, p1, 0.0) + jnp.where(oh2, p2, 0.0)
        out_ref[...] = jnp.zeros_like(out_ref)

    x = x_ref[...]
    h = jax.nn.silu(jnp.dot(x, w1_ref[0], preferred_element_type=jnp.float32))
    h = h * jnp.dot(x, w3_ref[0], preferred_element_type=jnp.float32)
    contrib = jnp.dot(h, w2_ref[0], preferred_element_type=jnp.float32)
    c = coef_ref[...]
    lane = jax.lax.broadcasted_iota(jnp.int32, c.shape, 1)
    coef = jnp.sum(jnp.where(lane == e, c, 0.0), axis=1, keepdims=True)
    out_ref[...] += coef * contrib


@jax.jit
def kernel(hidden_states, gate_w, w1, w2, w3):
    B, S, _ = hidden_states.shape
    T = B * S
    x = hidden_states.reshape(T, H)

    grid = (E, N_FT)
    out, logits = pl.pallas_call(
        _moe_kernel,
        grid=grid,
        in_specs=[
            pl.BlockSpec((T, H), lambda e, f: (0, 0)),
            pl.BlockSpec((H, E), lambda e, f: (0, 0)),
            pl.BlockSpec((1, H, FF_TILE), lambda e, f: (e, 0, f)),
            pl.BlockSpec((1, FF_TILE, H), lambda e, f: (e, f, 0)),
            pl.BlockSpec((1, H, FF_TILE), lambda e, f: (e, 0, f)),
        ],
        out_specs=[
            pl.BlockSpec((T, H), lambda e, f: (0, 0)),
            pl.BlockSpec((T, E), lambda e, f: (0, 0)),
        ],
        out_shape=[
            jax.ShapeDtypeStruct((T, H), jnp.float32),
            jax.ShapeDtypeStruct((T, E), jnp.float32),
        ],
        scratch_shapes=[pltpu.VMEM((T, E), jnp.float32)],
    )(x, gate_w, w1, w2, w3)

    return out.reshape(B, S, H), logits.reshape(B, S, E)
